# baseline (speedup 1.0000x reference)
import functools

import jax
import jax.numpy as jnp
from jax import lax
from jax.experimental import pallas as pl
from jax.experimental.pallas import tpu as pltpu

N_TOK = 12544
D_EMB = 64
K_CODES = 512
BN = 256


KB = 128


def _vq_body(z_hwc_ref, z_chw_ref, cb_ref, q_ref, loss_ref):
    z = z_hwc_ref[...]
    best_d = jnp.full((BN, 1), jnp.inf, jnp.float32)
    best_i = jnp.zeros((BN, 1), jnp.int32)
    for kb in range(K_CODES // KB):
        cbb = cb_ref[kb * KB:(kb + 1) * KB, :]
        s = lax.dot_general(z, cbb, (((1,), (1,)), ((), ())),
                            preferred_element_type=jnp.float32)
        c2 = jnp.sum(cbb * cbb, axis=1)
        d = c2[None, :] - 2.0 * s
        mb = jnp.min(d, axis=1, keepdims=True)
        iota = lax.broadcasted_iota(jnp.int32, d.shape, 1) + kb * KB
        ib = jnp.min(jnp.where(d == mb, iota, K_CODES),
                     axis=1, keepdims=True)
        take = mb < best_d
        best_i = jnp.where(take, ib, best_i)
        best_d = jnp.where(take, mb, best_d)
    q = jnp.zeros((BN, D_EMB), jnp.float32)
    for kb in range(K_CODES // KB):
        cbb = cb_ref[kb * KB:(kb + 1) * KB, :]
        iota = lax.broadcasted_iota(jnp.int32, (BN, KB), 1) + kb * KB
        onehot = (iota == best_i).astype(jnp.float32)
        q = q + lax.dot_general(onehot, cbb, (((1,), (0,)), ((), ())),
                                preferred_element_type=jnp.float32)
    q_ref[...] = q
    diff = q - z_chw_ref[...]
    part = jnp.sum(diff * diff)

    @pl.when(pl.program_id(0) == 0)
    def _():
        loss_ref[0, 0] = 0.0

    loss_ref[0, 0] += part


@jax.jit
def _vq(z_hwc, z_chw, codebook):
    grid = N_TOK // BN
    q, loss_sum = pl.pallas_call(
        _vq_body,
        grid=(grid,),
        in_specs=[
            pl.BlockSpec((BN, D_EMB), lambda i: (i, 0)),
            pl.BlockSpec((BN, D_EMB), lambda i: (i, 0)),
            pl.BlockSpec((K_CODES, D_EMB), lambda i: (0, 0)),
        ],
        out_specs=[
            pl.BlockSpec((BN, D_EMB), lambda i: (i, 0)),
            pl.BlockSpec(memory_space=pltpu.SMEM),
        ],
        out_shape=[
            jax.ShapeDtypeStruct((N_TOK, D_EMB), jnp.float32),
            jax.ShapeDtypeStruct((1, 1), jnp.float32),
        ],
    )(z_hwc, z_chw, codebook)
    return q, loss_sum


def _conv2d(x, w, b, stride, pad):
    y = lax.conv_general_dilated(x, w, (stride, stride),
                                 [(pad, pad), (pad, pad)],
                                 dimension_numbers=('NCHW', 'OIHW', 'NCHW'))
    return y + b[None, :, None, None]


def _conv_transpose2d(x, w, b, stride, pad):
    k = w.shape[2]
    w_f = jnp.flip(w, axis=(2, 3)).transpose(1, 0, 2, 3)
    y = lax.conv_general_dilated(x, w_f, (1, 1),
                                 [(k - 1 - pad, k - 1 - pad)] * 2,
                                 lhs_dilation=(stride, stride),
                                 dimension_numbers=('NCHW', 'OIHW', 'NCHW'))
    return y + b[None, :, None, None]


def kernel(x, enc_w1, enc_b1, enc_w2, enc_b2, enc_w3, enc_b3,
           proj_w, proj_b, codebook,
           dec_w1, dec_b1, dec_w2, dec_b2, dec_w3, dec_b3):
    h = jax.nn.relu(_conv2d(x, enc_w1, enc_b1, 2, 1))
    h = jax.nn.relu(_conv2d(h, enc_w2, enc_b2, 2, 1))
    z = _conv2d(h, enc_w3, enc_b3, 1, 1)
    z = _conv2d(z, proj_w, proj_b, 1, 0)

    z_hwc = jnp.transpose(z, (0, 2, 3, 1)).reshape(N_TOK, D_EMB)
    z_chw = z.reshape(N_TOK, D_EMB)
    q, loss_sum = _vq(z_hwc, z_chw, codebook)
    loss = loss_sum[0, 0] * (1.25 / (N_TOK * D_EMB))
    quantized = q.reshape(z.shape)

    d = jax.nn.relu(_conv_transpose2d(quantized, dec_w1, dec_b1, 1, 1))
    d = jax.nn.relu(_conv_transpose2d(d, dec_w2, dec_b2, 2, 1))
    x_recon = jax.nn.sigmoid(_conv_transpose2d(d, dec_w3, dec_b3, 2, 1))
    return (x_recon, loss)
